# Initial kernel scaffold; baseline (speedup 1.0000x reference)
import functools

import jax
import jax.numpy as jnp
from jax import lax
from jax.experimental import pallas as pl
from jax.experimental.pallas import tpu as pltpu
from jax.experimental.pallas import tpu_sc as plsc

NC = 2
NS = 16
NW = NC * NS

f32 = jnp.float32
i32 = jnp.int32


def _chunking(E):
    ept = E // NW
    assert ept * NW == E
    for ch in range(128, 0, -1):
        if ept % ch == 0:
            return ch, ept // ch
    raise ValueError(E)


def _sc_mesh():
    return plsc.VectorSubcoreMesh(
        core_axis_name="c", subcore_axis_name="s", num_cores=NC, num_subcores=NS
    )


@functools.cache
def _make_deg_kernel(N, E):
    CH, NCHUNK = _chunking(E)
    RPT = N // NS
    assert RPT * NS == N

    @functools.partial(
        pl.kernel,
        mesh=_sc_mesh(),
        out_type=(
            jax.ShapeDtypeStruct((NC, N, 16), f32),
            jax.ShapeDtypeStruct((NC, N, 16), f32),
        ),
        scratch_types=[
            pltpu.VMEM((NCHUNK, CH), i32),
            pltpu.VMEM((NCHUNK, CH), i32),
            pltpu.VMEM((CH, 16), f32),
            pltpu.VMEM((RPT, 16), f32),
            pltpu.VMEM_SHARED((N, 16), f32),
            pltpu.VMEM_SHARED((N, 16), f32),
        ],
    )
    def deg_kernel(src_hbm, dst_hbm, od_hbm, id_hbm, src_v, dst_v, ones_v,
                   zero_v, acc_od, acc_id):
        c = lax.axis_index("c")
        s = lax.axis_index("s")
        wid = c * NS + s

        def init_ones(r, _):
            ones_v[r] = jnp.ones((16,), f32)
            return 0

        lax.fori_loop(0, CH, init_ones, 0)

        def init_zero(r, _):
            zero_v[r] = jnp.zeros((16,), f32)
            return 0

        lax.fori_loop(0, RPT, init_zero, 0)

        pltpu.sync_copy(zero_v, acc_od.at[pl.ds(s * RPT, RPT)])
        pltpu.sync_copy(zero_v, acc_id.at[pl.ds(s * RPT, RPT)])
        plsc.subcore_barrier()

        pltpu.sync_copy(src_hbm.at[pl.ds(wid * NCHUNK, NCHUNK)], src_v)
        pltpu.sync_copy(dst_hbm.at[pl.ds(wid * NCHUNK, NCHUNK)], dst_v)

        def body(j, _):
            pltpu.sync_copy(ones_v, acc_od.at[src_v.at[j]], add=True)
            pltpu.sync_copy(ones_v, acc_id.at[dst_v.at[j]], add=True)
            return 0

        lax.fori_loop(0, NCHUNK, body, 0)
        plsc.subcore_barrier()

        pltpu.sync_copy(acc_od.at[pl.ds(s * RPT, RPT)],
                        od_hbm.at[c, pl.ds(s * RPT, RPT)])
        pltpu.sync_copy(acc_id.at[pl.ds(s * RPT, RPT)],
                        id_hbm.at[c, pl.ds(s * RPT, RPT)])

    return deg_kernel


@functools.cache
def _make_scatter_kernel(N, E, D):
    CH, NCHUNK = _chunking(E)
    RPT = N // NS
    assert RPT * NS == N and D % 16 == 0
    assert RPT % CH == 0

    @functools.partial(
        pl.kernel,
        mesh=_sc_mesh(),
        out_type=jax.ShapeDtypeStruct((NC, N, D), f32),
        scratch_types=[
            pltpu.VMEM((NCHUNK, CH), i32),
            pltpu.VMEM((NCHUNK, CH), i32),
            pltpu.VMEM((CH, D), f32),
            pltpu.VMEM_SHARED((N, D), f32),
        ],
    )
    def scatter_kernel(x_hbm, src_hbm, dst_hbm, out_hbm, src_v, dst_v, rows_v,
                       acc_sh):
        c = lax.axis_index("c")
        s = lax.axis_index("s")
        wid = c * NS + s

        def zrow(r, _):
            def zcol(k, _):
                rows_v[r, pl.ds(k * 16, 16)] = jnp.zeros((16,), f32)
                return 0

            return lax.fori_loop(0, D // 16, zcol, 0)

        lax.fori_loop(0, CH, zrow, 0)

        def zstripe(k, _):
            pltpu.sync_copy(rows_v, acc_sh.at[pl.ds(s * RPT + k * CH, CH)])
            return 0

        lax.fori_loop(0, RPT // CH, zstripe, 0)
        plsc.subcore_barrier()

        pltpu.sync_copy(src_hbm.at[pl.ds(wid * NCHUNK, NCHUNK)], src_v)
        pltpu.sync_copy(dst_hbm.at[pl.ds(wid * NCHUNK, NCHUNK)], dst_v)

        def body(j, _):
            pltpu.sync_copy(x_hbm.at[src_v.at[j]], rows_v)
            pltpu.sync_copy(rows_v, acc_sh.at[dst_v.at[j]], add=True)
            return 0

        lax.fori_loop(0, NCHUNK, body, 0)
        plsc.subcore_barrier()

        pltpu.sync_copy(acc_sh.at[pl.ds(s * RPT, RPT)],
                        out_hbm.at[c, pl.ds(s * RPT, RPT)])

    return scatter_kernel


def _stage_a(feats, W1, odp, idp):
    N, D = feats.shape
    H = W1.shape[1]

    def body(f_ref, w_ref, od_ref, id_ref, y1_ref, ns_ref, nd_ref):
        od = od_ref[0, :, 0:1] + od_ref[1, :, 0:1]
        ind = id_ref[0, :, 0:1] + id_ref[1, :, 0:1]
        ns = lax.rsqrt(jnp.maximum(od, 1.0))
        nd = lax.rsqrt(jnp.maximum(ind, 1.0))
        z = jnp.dot(f_ref[...], w_ref[...], preferred_element_type=f32)
        y1_ref[...] = z * ns
        ns_ref[...] = ns
        nd_ref[...] = nd

    return pl.pallas_call(
        body,
        out_shape=(
            jax.ShapeDtypeStruct((N, H), f32),
            jax.ShapeDtypeStruct((N, 1), f32),
            jax.ShapeDtypeStruct((N, 1), f32),
        ),
    )(feats, W1, odp, idp)


def _stage_b(aggp, nd, b1, W2, ns):
    _, N, H = aggp.shape

    def body(a_ref, nd_ref, b_ref, w_ref, ns_ref, h1_ref, y2_ref):
        agg = a_ref[0] + a_ref[1]
        h1 = jnp.maximum(agg * nd_ref[...] + b_ref[...], 0.0)
        h1_ref[...] = h1
        y2_ref[...] = jnp.dot(h1, w_ref[...], preferred_element_type=f32) * ns_ref[...]

    return pl.pallas_call(
        body,
        out_shape=(
            jax.ShapeDtypeStruct((N, H), f32),
            jax.ShapeDtypeStruct((N, W2.shape[1]), f32),
        ),
    )(aggp, nd, b1, W2, ns)


def _stage_c(aggp, nd, b2, h1, w_top, w_bot):
    _, N, H = aggp.shape
    O = w_top.shape[1]

    def body(a_ref, nd_ref, b_ref, h1_ref, wt_ref, wb_ref, g_ref):
        h2 = jnp.maximum((a_ref[0] + a_ref[1]) * nd_ref[...] + b_ref[...], 0.0)
        g_ref[...] = jnp.dot(h1_ref[...], wt_ref[...], preferred_element_type=f32) + jnp.dot(
            h2, wb_ref[...], preferred_element_type=f32)

    return pl.pallas_call(
        body,
        out_shape=jax.ShapeDtypeStruct((N, O), f32),
    )(aggp, nd, b2, h1, w_top, w_bot)


def _stage_d(jp, bout):
    _, N, O = jp.shape

    def body(j_ref, b_ref, o_ref):
        o_ref[...] = j_ref[0] + j_ref[1] + b_ref[...]

    return pl.pallas_call(
        body,
        out_shape=jax.ShapeDtypeStruct((N, O), f32),
    )(jp, bout)


def kernel(feats, edge_index, W1, b1, W2, b2, Wout, bout):
    N, D = feats.shape
    E = edge_index.shape[1]
    H = W1.shape[1]
    CH, NCHUNK = _chunking(E)

    src = edge_index[0].reshape(NW * NCHUNK, CH)
    dst = edge_index[1].reshape(NW * NCHUNK, CH)

    deg_k = _make_deg_kernel(N, E)
    scat_h = _make_scatter_kernel(N, E, H)

    odp, idp = deg_k(src, dst)
    y1, ns, nd = _stage_a(feats, W1, odp, idp)
    agg1 = scat_h(y1, src, dst)
    h1, y2 = _stage_b(agg1, nd, b1.reshape(1, H), W2, ns)
    agg2 = scat_h(y2, src, dst)
    g = _stage_c(agg2, nd, b2.reshape(1, H), h1, Wout[:H], Wout[H:])
    jp = scat_h(g, src, dst)
    return _stage_d(jp, bout.reshape(1, -1))


# speedup vs baseline: 5.3207x; 5.3207x over previous
import functools

import jax
import jax.numpy as jnp
from jax import lax
from jax.experimental import pallas as pl
from jax.experimental.pallas import tpu as pltpu
from jax.experimental.pallas import tpu_sc as plsc

NC = 2
NS = 16
NW = NC * NS

f32 = jnp.float32
i32 = jnp.int32


def _chunking(E):
    ept = E // NW
    assert ept * NW == E
    for ch in range(128, 0, -1):
        if ept % ch == 0:
            return ch, ept // ch
    raise ValueError(E)


def _sc_mesh():
    return plsc.VectorSubcoreMesh(
        core_axis_name="c", subcore_axis_name="s", num_cores=NC, num_subcores=NS
    )


def _pad_rows(N):
    q = NS * 64
    return ((N + q - 1) // q) * q


GR = 8
ZC = 80


@functools.cache
def _make_deg_kernel(N, E):
    CH, NCHUNK = _chunking(E)
    NP = _pad_rows(N)
    RPT = NP // NS
    assert NCHUNK % GR == 0 and RPT % ZC == 0

    @functools.partial(
        pl.kernel,
        mesh=_sc_mesh(),
        out_type=(
            jax.ShapeDtypeStruct((NC, NP, 16), f32),
            jax.ShapeDtypeStruct((NC, NP, 16), f32),
        ),
        scratch_types=[
            pltpu.VMEM((GR, CH), i32),
            pltpu.VMEM((GR, CH), i32),
            pltpu.VMEM((CH, 16), f32),
            pltpu.VMEM_SHARED((NP, 16), f32),
            pltpu.VMEM_SHARED((NP, 16), f32),
        ],
    )
    def deg_kernel(src_hbm, dst_hbm, ones_hbm, zeros_hbm, od_hbm, id_hbm,
                   src_v, dst_v, ones_v, acc_od, acc_id):
        c = lax.axis_index("c")
        s = lax.axis_index("s")
        wid = c * NS + s

        pltpu.sync_copy(ones_hbm, ones_v)

        def zstripe(k, _):
            pltpu.sync_copy(zeros_hbm.at[pl.ds(0, ZC)],
                            acc_od.at[pl.ds(s * RPT + k * ZC, ZC)])
            pltpu.sync_copy(zeros_hbm.at[pl.ds(0, ZC)],
                            acc_id.at[pl.ds(s * RPT + k * ZC, ZC)])
            return 0

        lax.fori_loop(0, RPT // ZC, zstripe, 0)
        plsc.subcore_barrier()

        def outer(jj, _):
            pltpu.sync_copy(src_hbm.at[pl.ds(wid * NCHUNK + jj * GR, GR)], src_v)
            pltpu.sync_copy(dst_hbm.at[pl.ds(wid * NCHUNK + jj * GR, GR)], dst_v)

            def body(j, _):
                pltpu.sync_copy(ones_v, acc_od.at[src_v.at[j]], add=True)
                pltpu.sync_copy(ones_v, acc_id.at[dst_v.at[j]], add=True)
                return 0

            return lax.fori_loop(0, GR, body, 0)

        lax.fori_loop(0, NCHUNK // GR, outer, 0)
        plsc.subcore_barrier()

        pltpu.sync_copy(acc_od.at[pl.ds(s * RPT, RPT)],
                        od_hbm.at[c, pl.ds(s * RPT, RPT)])
        pltpu.sync_copy(acc_id.at[pl.ds(s * RPT, RPT)],
                        id_hbm.at[c, pl.ds(s * RPT, RPT)])

    return deg_kernel


@functools.cache
def _make_scatter_kernel(N, E, D):
    CH, NCHUNK = _chunking(E)
    NP = _pad_rows(N)
    RPT = NP // NS
    assert D % 16 == 0
    assert RPT % ZC == 0 and ZC <= CH and NCHUNK % GR == 0

    @functools.partial(
        pl.kernel,
        mesh=_sc_mesh(),
        out_type=jax.ShapeDtypeStruct((NC, NP, D), f32),
        scratch_types=[
            pltpu.VMEM((GR, CH), i32),
            pltpu.VMEM((GR, CH), i32),
            pltpu.VMEM((CH, D), f32),
            pltpu.VMEM_SHARED((NP, D), f32),
        ],
    )
    def scatter_kernel(x_hbm, src_hbm, dst_hbm, zeros_hbm, out_hbm, src_v,
                       dst_v, rows_v, acc_sh):
        c = lax.axis_index("c")
        s = lax.axis_index("s")
        wid = c * NS + s

        def zstripe(k, _):
            pltpu.sync_copy(zeros_hbm.at[pl.ds(0, ZC)],
                            acc_sh.at[pl.ds(s * RPT + k * ZC, ZC)])
            return 0

        lax.fori_loop(0, RPT // ZC, zstripe, 0)
        plsc.subcore_barrier()

        def outer(jj, _):
            pltpu.sync_copy(src_hbm.at[pl.ds(wid * NCHUNK + jj * GR, GR)], src_v)
            pltpu.sync_copy(dst_hbm.at[pl.ds(wid * NCHUNK + jj * GR, GR)], dst_v)

            def body(j, _):
                pltpu.sync_copy(x_hbm.at[src_v.at[j]], rows_v)
                pltpu.sync_copy(rows_v, acc_sh.at[dst_v.at[j]], add=True)
                return 0

            return lax.fori_loop(0, GR, body, 0)

        lax.fori_loop(0, NCHUNK // GR, outer, 0)
        plsc.subcore_barrier()

        pltpu.sync_copy(acc_sh.at[pl.ds(s * RPT, RPT)],
                        out_hbm.at[c, pl.ds(s * RPT, RPT)])

    return scatter_kernel


def _stage_a(feats, W1, odp, idp):
    N, D = feats.shape
    H = W1.shape[1]

    def body(f_ref, w_ref, od_ref, id_ref, y1_ref, ns_ref, nd_ref):
        od = od_ref[0, 0:N, 0:1] + od_ref[1, 0:N, 0:1]
        ind = id_ref[0, 0:N, 0:1] + id_ref[1, 0:N, 0:1]
        ns = lax.rsqrt(jnp.maximum(od, 1.0))
        nd = lax.rsqrt(jnp.maximum(ind, 1.0))
        z = jnp.dot(f_ref[...], w_ref[...], preferred_element_type=f32)
        y1_ref[...] = z * ns
        ns_ref[...] = ns
        nd_ref[...] = nd

    return pl.pallas_call(
        body,
        out_shape=(
            jax.ShapeDtypeStruct((N, H), f32),
            jax.ShapeDtypeStruct((N, 1), f32),
            jax.ShapeDtypeStruct((N, 1), f32),
        ),
    )(feats, W1, odp, idp)


def _stage_b(aggp, nd, b1, W2, ns):
    _, _, H = aggp.shape
    N = nd.shape[0]

    def body(a_ref, nd_ref, b_ref, w_ref, ns_ref, h1_ref, y2_ref):
        agg = a_ref[0, 0:N] + a_ref[1, 0:N]
        h1 = jnp.maximum(agg * nd_ref[...] + b_ref[...], 0.0)
        h1_ref[...] = h1
        y2_ref[...] = jnp.dot(h1, w_ref[...], preferred_element_type=f32) * ns_ref[...]

    return pl.pallas_call(
        body,
        out_shape=(
            jax.ShapeDtypeStruct((N, H), f32),
            jax.ShapeDtypeStruct((N, W2.shape[1]), f32),
        ),
    )(aggp, nd, b1, W2, ns)


def _stage_c(aggp, nd, b2, h1, w_top, w_bot):
    N = nd.shape[0]
    O = w_top.shape[1]

    def body(a_ref, nd_ref, b_ref, h1_ref, wt_ref, wb_ref, g_ref):
        h2 = jnp.maximum((a_ref[0, 0:N] + a_ref[1, 0:N]) * nd_ref[...] + b_ref[...], 0.0)
        g_ref[...] = jnp.dot(h1_ref[...], wt_ref[...], preferred_element_type=f32) + jnp.dot(
            h2, wb_ref[...], preferred_element_type=f32)

    return pl.pallas_call(
        body,
        out_shape=jax.ShapeDtypeStruct((N, O), f32),
    )(aggp, nd, b2, h1, w_top, w_bot)


def _stage_d(jp, bout, N):
    O = jp.shape[2]

    def body(j_ref, b_ref, o_ref):
        o_ref[...] = j_ref[0, 0:N] + j_ref[1, 0:N] + b_ref[...]

    return pl.pallas_call(
        body,
        out_shape=jax.ShapeDtypeStruct((N, O), f32),
    )(jp, bout)


def kernel(feats, edge_index, W1, b1, W2, b2, Wout, bout):
    N, D = feats.shape
    E = edge_index.shape[1]
    H = W1.shape[1]
    CH, NCHUNK = _chunking(E)

    src = edge_index[0].reshape(NW * NCHUNK, CH)
    dst = edge_index[1].reshape(NW * NCHUNK, CH)

    scat_h = _make_scatter_kernel(N, E, H)

    zerosH = jnp.zeros((ZC, H), f32)
    onesN = jnp.ones((N, H), f32)

    odp = scat_h(onesN, src, src, zerosH)
    idp = scat_h(onesN, dst, dst, zerosH)
    y1, ns, nd = _stage_a(feats, W1, odp, idp)
    agg1 = scat_h(y1, src, dst, zerosH)
    h1, y2 = _stage_b(agg1, nd, b1.reshape(1, H), W2, ns)
    agg2 = scat_h(y2, src, dst, zerosH)
    g = _stage_c(agg2, nd, b2.reshape(1, H), h1, Wout[:H], Wout[H:])
    jp = scat_h(g, src, dst, zerosH)
    return _stage_d(jp, bout.reshape(1, -1), N)


# speedup vs baseline: 6.9848x; 1.3128x over previous
import functools

import jax
import jax.numpy as jnp
from jax import lax
from jax.experimental import pallas as pl
from jax.experimental.pallas import tpu as pltpu
from jax.experimental.pallas import tpu_sc as plsc

NC = 2
NS = 16
NW = NC * NS

f32 = jnp.float32
i32 = jnp.int32


def _chunking(E):
    ept = E // NW
    assert ept * NW == E
    for ch in range(128, 0, -1):
        if ept % ch == 0:
            return ch, ept // ch
    raise ValueError(E)


def _sc_mesh():
    return plsc.VectorSubcoreMesh(
        core_axis_name="c", subcore_axis_name="s", num_cores=NC, num_subcores=NS
    )


def _pad_rows(N):
    q = NS * 64
    return ((N + q - 1) // q) * q


GR = 8
ZC = 80


@functools.cache
def _make_scatter_kernel(N, E, D):
    CH, NCHUNK = _chunking(E)
    NP = _pad_rows(N)
    RPT = NP // NS
    assert D % 16 == 0
    assert RPT % ZC == 0 and ZC <= CH and NCHUNK % GR == 0

    @functools.partial(
        pl.kernel,
        mesh=_sc_mesh(),
        out_type=jax.ShapeDtypeStruct((NC, NP, D), f32),
        scratch_types=[
            pltpu.VMEM((GR, CH), i32),
            pltpu.VMEM((GR, CH), i32),
            pltpu.VMEM((CH, D), f32),
            pltpu.VMEM((CH, D), f32),
            pltpu.VMEM_SHARED((NP, D), f32),
            pltpu.SemaphoreType.DMA,
            pltpu.SemaphoreType.DMA,
        ],
    )
    def scatter_kernel(x_hbm, src_hbm, dst_hbm, zeros_hbm, out_hbm, src_v,
                       dst_v, rows_a, rows_b, acc_sh, sem_a, sem_b):
        c = lax.axis_index("c")
        s = lax.axis_index("s")
        wid = c * NS + s

        def zstripe(k, _):
            pltpu.sync_copy(zeros_hbm.at[pl.ds(0, ZC)],
                            acc_sh.at[pl.ds(s * RPT + k * ZC, ZC)])
            return 0

        lax.fori_loop(0, RPT // ZC, zstripe, 0)
        plsc.subcore_barrier()

        pltpu.sync_copy(src_hbm.at[pl.ds(wid * NCHUNK, GR)], src_v)
        pltpu.sync_copy(dst_hbm.at[pl.ds(wid * NCHUNK, GR)], dst_v)
        pltpu.async_copy(x_hbm.at[src_v.at[0]], rows_a, sem_a)

        def outer(g, _):
            j = g * 2
            ja = j % GR
            jb = ja + 1
            more = j + 2 < NCHUNK
            wrap = jnp.logical_and(more, jb == GR - 1)

            pltpu.make_async_copy(x_hbm.at[src_v.at[ja]], rows_a, sem_a).wait()
            pltpu.async_copy(x_hbm.at[src_v.at[jb]], rows_b, sem_b)
            pltpu.sync_copy(rows_a, acc_sh.at[dst_v.at[ja]], add=True)

            pltpu.make_async_copy(x_hbm.at[src_v.at[jb]], rows_b, sem_b).wait()

            @pl.when(wrap)
            def _refresh_src():
                off = pl.multiple_of(wid * NCHUNK + j + 2, GR)
                pltpu.sync_copy(src_hbm.at[pl.ds(off, GR)], src_v)

            @pl.when(more)
            def _start_next():
                pltpu.async_copy(x_hbm.at[src_v.at[(j + 2) % GR]], rows_a,
                                 sem_a)

            pltpu.sync_copy(rows_b, acc_sh.at[dst_v.at[jb]], add=True)

            @pl.when(wrap)
            def _refresh_dst():
                off = pl.multiple_of(wid * NCHUNK + j + 2, GR)
                pltpu.sync_copy(dst_hbm.at[pl.ds(off, GR)], dst_v)

            return 0

        lax.fori_loop(0, NCHUNK // 2, outer, 0)
        plsc.subcore_barrier()

        pltpu.sync_copy(acc_sh.at[pl.ds(s * RPT, RPT)],
                        out_hbm.at[c, pl.ds(s * RPT, RPT)])

    return scatter_kernel


def _stage_a(feats, W1, odp, idp):
    N, D = feats.shape
    H = W1.shape[1]

    def body(f_ref, w_ref, od_ref, id_ref, y1_ref, ns_ref, nd_ref):
        od = od_ref[0, 0:N, 0:1] + od_ref[1, 0:N, 0:1]
        ind = id_ref[0, 0:N, 0:1] + id_ref[1, 0:N, 0:1]
        ns = lax.rsqrt(jnp.maximum(od, 1.0))
        nd = lax.rsqrt(jnp.maximum(ind, 1.0))
        z = jnp.dot(f_ref[...], w_ref[...], preferred_element_type=f32)
        y1_ref[...] = z * ns
        ns_ref[...] = ns
        nd_ref[...] = nd

    return pl.pallas_call(
        body,
        out_shape=(
            jax.ShapeDtypeStruct((N, H), f32),
            jax.ShapeDtypeStruct((N, 1), f32),
            jax.ShapeDtypeStruct((N, 1), f32),
        ),
    )(feats, W1, odp, idp)


def _stage_b(aggp, nd, b1, W2, ns):
    _, _, H = aggp.shape
    N = nd.shape[0]

    def body(a_ref, nd_ref, b_ref, w_ref, ns_ref, h1_ref, y2_ref):
        agg = a_ref[0, 0:N] + a_ref[1, 0:N]
        h1 = jnp.maximum(agg * nd_ref[...] + b_ref[...], 0.0)
        h1_ref[...] = h1
        y2_ref[...] = jnp.dot(h1, w_ref[...], preferred_element_type=f32) * ns_ref[...]

    return pl.pallas_call(
        body,
        out_shape=(
            jax.ShapeDtypeStruct((N, H), f32),
            jax.ShapeDtypeStruct((N, W2.shape[1]), f32),
        ),
    )(aggp, nd, b1, W2, ns)


def _stage_c(aggp, nd, b2, h1, w_top, w_bot):
    N = nd.shape[0]
    O = w_top.shape[1]

    def body(a_ref, nd_ref, b_ref, h1_ref, wt_ref, wb_ref, g_ref):
        h2 = jnp.maximum((a_ref[0, 0:N] + a_ref[1, 0:N]) * nd_ref[...] + b_ref[...], 0.0)
        g_ref[...] = jnp.dot(h1_ref[...], wt_ref[...], preferred_element_type=f32) + jnp.dot(
            h2, wb_ref[...], preferred_element_type=f32)

    return pl.pallas_call(
        body,
        out_shape=jax.ShapeDtypeStruct((N, O), f32),
    )(aggp, nd, b2, h1, w_top, w_bot)


def _stage_d(jp, bout, N):
    O = jp.shape[2]

    def body(j_ref, b_ref, o_ref):
        o_ref[...] = j_ref[0, 0:N] + j_ref[1, 0:N] + b_ref[...]

    return pl.pallas_call(
        body,
        out_shape=jax.ShapeDtypeStruct((N, O), f32),
    )(jp, bout)


def kernel(feats, edge_index, W1, b1, W2, b2, Wout, bout):
    N, D = feats.shape
    E = edge_index.shape[1]
    H = W1.shape[1]
    CH, NCHUNK = _chunking(E)

    src = edge_index[0].reshape(NW * NCHUNK, CH)
    dst = edge_index[1].reshape(NW * NCHUNK, CH)

    scat_h = _make_scatter_kernel(N, E, H)

    zerosH = jnp.zeros((ZC, H), f32)
    onesN = jnp.ones((N, H), f32)

    odp = scat_h(onesN, src, src, zerosH)
    idp = scat_h(onesN, dst, dst, zerosH)
    y1, ns, nd = _stage_a(feats, W1, odp, idp)
    agg1 = scat_h(y1, src, dst, zerosH)
    h1, y2 = _stage_b(agg1, nd, b1.reshape(1, H), W2, ns)
    agg2 = scat_h(y2, src, dst, zerosH)
    g = _stage_c(agg2, nd, b2.reshape(1, H), h1, Wout[:H], Wout[H:])
    jp = scat_h(g, src, dst, zerosH)
    return _stage_d(jp, bout.reshape(1, -1), N)
